# baseline (speedup 1.0000x reference)
import functools

import jax
import jax.numpy as jnp
from jax import lax
from jax.experimental import pallas as pl
from jax.experimental.pallas import tpu as pltpu
from jax.experimental.pallas import tpu_sc as plsc

NUM_CORES = 2
NUM_SUBCORES = 16
NUM_WORKERS = NUM_CORES * NUM_SUBCORES


def kernel(x, embed_table):
    x = x.astype(jnp.int32)
    (B,) = x.shape
    V, D = embed_table.shape
    b_per_w = B // NUM_WORKERS
    C = 32
    n_chunks = b_per_w // C

    mesh = plsc.VectorSubcoreMesh(core_axis_name="c", subcore_axis_name="s")

    @functools.partial(
        pl.kernel,
        mesh=mesh,
        out_type=jax.ShapeDtypeStruct((B, D), jnp.float32),
        scratch_types=[
            pltpu.VMEM((b_per_w,), jnp.int32),
            pltpu.VMEM((C, D), jnp.float32),
            pltpu.SemaphoreType.DMA,
        ],
    )
    def sc_gather(table_hbm, idx_hbm, out_hbm, idx_v, rows_v, sem):
        wid = lax.axis_index("s") * NUM_CORES + lax.axis_index("c")
        base = wid * b_per_w
        pltpu.sync_copy(idx_hbm.at[pl.ds(base, b_per_w)], idx_v)

        def body(c, carry):
            off = c * C
            pltpu.async_copy(
                table_hbm.at[idx_v.at[pl.ds(off, C)]], rows_v, sem
            ).wait()
            pltpu.sync_copy(rows_v, out_hbm.at[pl.ds(base + off, C)])
            return carry

        lax.fori_loop(0, n_chunks, body, 0)

    return sc_gather(embed_table, x)


# speedup vs baseline: 1.0077x; 1.0077x over previous
import functools

import jax
import jax.numpy as jnp
from jax import lax
from jax.experimental import pallas as pl
from jax.experimental.pallas import tpu as pltpu
from jax.experimental.pallas import tpu_sc as plsc

NUM_CORES = 2
NUM_SUBCORES = 16
NUM_WORKERS = NUM_CORES * NUM_SUBCORES


def kernel(x, embed_table):
    x = x.astype(jnp.int32)
    (B,) = x.shape
    V, D = embed_table.shape
    b_per_w = B // NUM_WORKERS
    C = 16
    n_chunks = b_per_w // C

    mesh = plsc.VectorSubcoreMesh(core_axis_name="c", subcore_axis_name="s")

    @functools.partial(
        pl.kernel,
        mesh=mesh,
        out_type=jax.ShapeDtypeStruct((B, D), jnp.float32),
        scratch_types=[
            pltpu.VMEM((b_per_w,), jnp.int32),
            pltpu.VMEM((C, D), jnp.float32),
            pltpu.VMEM((C, D), jnp.float32),
            pltpu.SemaphoreType.DMA,
            pltpu.SemaphoreType.DMA,
            pltpu.SemaphoreType.DMA,
            pltpu.SemaphoreType.DMA,
        ],
    )
    def sc_gather(table_hbm, idx_hbm, out_hbm, idx_v, rows0, rows1,
                  gsem0, gsem1, wsem0, wsem1):
        wid = lax.axis_index("s") * NUM_CORES + lax.axis_index("c")
        base = wid * b_per_w
        pltpu.sync_copy(idx_hbm.at[pl.ds(base, b_per_w)], idx_v)

        rows = (rows0, rows1)
        gsem = (gsem0, gsem1)
        wsem = (wsem0, wsem1)

        def fire_gather(c):
            b = c % 2
            return pltpu.async_copy(
                table_hbm.at[idx_v.at[pl.ds(c * C, C)]], rows[b], gsem[b])

        def fire_write(c):
            b = c % 2
            return pltpu.async_copy(
                rows[b], out_hbm.at[pl.ds(base + c * C, C)], wsem[b])

        gh = [None] * n_chunks
        wh = [None] * n_chunks
        gh[0] = fire_gather(0)
        gh[1] = fire_gather(1)
        for c in range(n_chunks):
            gh[c].wait()
            wh[c] = fire_write(c)
            nxt = c + 1
            if 2 <= nxt < n_chunks:
                wh[nxt - 2].wait()
                gh[nxt] = fire_gather(nxt)
        wh[n_chunks - 2].wait()
        wh[n_chunks - 1].wait()

    return sc_gather(embed_table, x)


# speedup vs baseline: 4.7606x; 4.7243x over previous
import functools

import jax
import jax.numpy as jnp
from jax import lax
from jax.experimental import pallas as pl
from jax.experimental.pallas import tpu as pltpu
from jax.experimental.pallas import tpu_sc as plsc

NUM_CORES = 2
NUM_SUBCORES = 16
NUM_WORKERS = NUM_CORES * NUM_SUBCORES


def kernel(x, embed_table):
    x = x.astype(jnp.int32)
    (B,) = x.shape
    V, D = embed_table.shape
    b_per_w = B // NUM_WORKERS
    K = 8
    n_groups = b_per_w // K

    mesh = plsc.VectorSubcoreMesh(core_axis_name="c", subcore_axis_name="s")

    @functools.partial(
        pl.kernel,
        mesh=mesh,
        out_type=jax.ShapeDtypeStruct((B, D), jnp.float32),
        scratch_types=[
            pltpu.SMEM((b_per_w,), jnp.int32),
            pltpu.VMEM_SHARED((NUM_WORKERS, b_per_w), jnp.int32),
            pltpu.VMEM((V, D), jnp.float32),
            pltpu.SemaphoreType.DMA,
        ],
    )
    def sc_lookup(table_hbm, idx_hbm, out_hbm, idx_s, idx_v, table_v, sem):
        wid = lax.axis_index("s") * NUM_CORES + lax.axis_index("c")
        base = wid * b_per_w
        pltpu.sync_copy(table_hbm, table_v)
        pltpu.sync_copy(idx_hbm.at[pl.ds(base, b_per_w)], idx_v.at[wid])
        pltpu.sync_copy(idx_v.at[wid], idx_s)

        def fire(r):
            pltpu.async_copy(table_v.at[idx_s[r]], out_hbm.at[base + r], sem)

        def drain_one():
            pltpu.make_async_copy(
                table_hbm.at[0], out_hbm.at[base], sem
            ).wait()

        for j in range(K):
            fire(j)

        def body(g, carry):
            for j in range(K):
                fire(g * K + j)
            for j in range(K):
                drain_one()
            return carry

        lax.fori_loop(1, n_groups, body, 0)
        for j in range(K):
            drain_one()

    return sc_lookup(embed_table, x)
